# baseline (speedup 1.0000x reference)
import functools

import jax
import jax.numpy as jnp
from jax import lax
from jax.experimental import pallas as pl
from jax.experimental.pallas import tpu as pltpu
from jax.experimental.pallas import tpu_sc as plsc

F32 = jnp.float32
I32 = jnp.int32

_NC = 2
_NS = 16
_NW = _NC * _NS
_LANES = 16
_CHUNK = 128


def _fast_rsqrt(d):
    i = plsc.bitcast(d, I32)
    i = jnp.int32(0x5F3759DF) - (i >> 1)
    y = plsc.bitcast(i, F32)
    for _ in range(3):
        y = y * (1.5 - 0.5 * d * y * y)
    return y


def _make_sc_kernel(npad, cht_a, cht_c, rows_t):
    mesh = plsc.VectorSubcoreMesh(core_axis_name="c", subcore_axis_name="s")
    lane_iota = lambda: lax.iota(I32, _LANES)

    @functools.partial(
        pl.kernel,
        out_type=(
            jax.ShapeDtypeStruct((_NC, _LANES, npad), F32),
            jax.ShapeDtypeStruct((_NC, npad, _LANES), F32),
        ),
        mesh=mesh,
        compiler_params=pltpu.CompilerParams(use_tc_tiling_on_sc=False,
                                             needs_layout_passes=False),
        scratch_types=[
            pltpu.VMEM((cht_a, _CHUNK), I32),
            pltpu.VMEM((cht_c, _CHUNK), I32),
            pltpu.VMEM((_CHUNK,), F32),
            pltpu.VMEM((_LANES,), F32),
            pltpu.VMEM((8, _LANES), F32),
            pltpu.VMEM((_LANES, rows_t), F32),
            pltpu.VMEM((rows_t,), F32),
            pltpu.VMEM((rows_t,), F32),
            pltpu.VMEM((rows_t, _LANES), F32),
            pltpu.VMEM((_LANES, rows_t), F32),
            pltpu.VMEM((_CHUNK, _LANES), F32),
            pltpu.VMEM((_CHUNK, _LANES), F32),
            pltpu.VMEM((_CHUNK, _LANES), F32),
            pltpu.VMEM((_CHUNK, _LANES), F32),
            pltpu.VMEM_SHARED((npad,), F32),
            pltpu.VMEM_SHARED((npad, _LANES), F32),
            pltpu.SemaphoreType.DMA,
            pltpu.SemaphoreType.DMA,
            pltpu.SemaphoreType.DMA,
            pltpu.SemaphoreType.DMA,
            pltpu.SemaphoreType.DMA,
            pltpu.SemaphoreType.DMA,
            pltpu.SemaphoreType.DMA,
            pltpu.SemaphoreType.DMA,
            pltpu.SemaphoreType.DMA,
        ],
    )
    def sc_kernel(dst_h, src_h, x_h, g_h, xpo_h,
                  dst_v, src_v, ones_v, z1, z8, x_v, hist_v, dinv_v, xp_v,
                  gt_v, rb0, rb1, rb2, rb3, hist_sh, g_sh,
                  sem_a, g0, g1, g2, g3, s0, s1, s2, s3):
        c = lax.axis_index("c")
        s = lax.axis_index("s")
        for i in range(_CHUNK // _LANES):
            ones_v[pl.ds(i * _LANES, _LANES)] = jnp.ones((_LANES,), F32)
        z1[pl.ds(0, _LANES)] = jnp.zeros((_LANES,), F32)
        for i in range(8):
            z8[i, :] = jnp.zeros((_LANES,), F32)

        def zero_hist(j, _):
            pltpu.sync_copy(
                z1, hist_sh.at[pl.ds(s * rows_t + j * _LANES, _LANES)])
            return 0

        lax.fori_loop(0, rows_t // _LANES, zero_hist, 0)
        plsc.subcore_barrier()

        scope_a = jax.named_scope("phaseA")
        scope_a.__enter__()
        pltpu.sync_copy(dst_h.at[s], dst_v)
        n_grp = -(-cht_a // 8)

        def hist_fire(j, _):
            pltpu.async_copy(ones_v, hist_sh.at[dst_v.at[j]], sem_a,
                             add=True)
            return 0

        lax.fori_loop(0, cht_a, hist_fire, 0)

        def hist_drain(j, _):
            pltpu.make_async_copy(
                ones_v, hist_sh.at[dst_v.at[0]], sem_a).wait()
            return 0

        lax.fori_loop(0, cht_a, hist_drain, 0)
        plsc.subcore_barrier()

        scope_a.__exit__(None, None, None)
        scope_b = jax.named_scope("phaseB")
        scope_b.__enter__()
        pltpu.sync_copy(hist_sh.at[pl.ds(s * rows_t, rows_t)], hist_v)
        pltpu.sync_copy(x_h.at[:, pl.ds(s * rows_t, rows_t)], x_v)

        def brow(q, _):
            d = hist_v[pl.ds(q * _LANES, _LANES)] + 1.0
            dv = _fast_rsqrt(d)
            dinv_v[pl.ds(q * _LANES, _LANES)] = dv
            idx_row = q * _LANES + lane_iota()
            for ch in range(_LANES):
                idx_col = jnp.full((_LANES,), ch, I32)
                xcol = x_v[ch, pl.ds(q * _LANES, _LANES)]
                plsc.store_scatter(xp_v, [idx_row, idx_col], xcol * dv)
            return 0

        lax.fori_loop(0, rows_t // _LANES, brow, 0)
        pltpu.sync_copy(xp_v, xpo_h.at[c, pl.ds(s * rows_t, rows_t)])

        @pl.when(c == 0)
        def _():
            pltpu.sync_copy(xp_v, g_sh.at[pl.ds(s * rows_t, rows_t)])

        @pl.when(c == 1)
        def _():
            def zg(j, _):
                pltpu.sync_copy(z8, g_sh.at[pl.ds(s * rows_t + j * 8, 8)])
                return 0

            lax.fori_loop(0, rows_t // 8, zg, 0)

        plsc.subcore_barrier()

        scope_b.__exit__(None, None, None)
        scope_c = jax.named_scope("phaseC")
        scope_c.__enter__()
        pltpu.sync_copy(src_h.at[c, s], src_v)
        coff = c * cht_c
        bufs = (rb0, rb1, rb2, rb3)
        gsems = (g0, g1, g2, g3)
        ssems = (s0, s1, s2, s3)
        xp_t = xpo_h.at[c]
        for k in range(4):
            pltpu.async_copy(xp_t.at[src_v.at[k]], bufs[k], gsems[k])

        def edge_grp(gi_, _):
            for k in range(4):
                j = gi_ * 4 + k
                pltpu.make_async_copy(
                    xp_t.at[src_v.at[j]], bufs[k], gsems[k]).wait()
                pltpu.async_copy(bufs[k], g_sh.at[dst_v.at[coff + j]],
                                 ssems[k], add=True)
            for k in range(4):
                jn = gi_ * 4 + k + 4

                @pl.when(jn < cht_c)
                def _():
                    pltpu.make_async_copy(
                        bufs[k], g_sh.at[dst_v.at[coff + jn - 4]],
                        ssems[k]).wait()
                    pltpu.async_copy(xp_t.at[src_v.at[jn]], bufs[k],
                                     gsems[k])

            return 0

        lax.fori_loop(0, cht_c // 4, edge_grp, 0)
        for k in range(4):
            pltpu.make_async_copy(
                bufs[k], g_sh.at[dst_v.at[coff + cht_c - 4 + k]],
                ssems[k]).wait()
        plsc.subcore_barrier()

        scope_c.__exit__(None, None, None)
        scope_d = jax.named_scope("phaseD")
        scope_d.__enter__()
        pltpu.sync_copy(g_sh.at[pl.ds(s * rows_t, rows_t)], xp_v)

        def drow(q, _):
            dvv = dinv_v[pl.ds(q * _LANES, _LANES)]
            idx_row = q * _LANES + lane_iota()
            for ch in range(_LANES):
                idx_col = jnp.full((_LANES,), ch, I32)
                col = plsc.load_gather(xp_v, [idx_row, idx_col]) * dvv
                gt_v[ch, pl.ds(q * _LANES, _LANES)] = col
            return 0

        lax.fori_loop(0, rows_t // _LANES, drow, 0)
        pltpu.sync_copy(gt_v, g_h.at[c, :, pl.ds(s * rows_t, rows_t)])
        scope_d.__exit__(None, None, None)

    return sc_kernel


def _gru_body(T, HT, g_ref, g2_ref, wih_ref, whh_ref, bihz_ref,
              bhh_ref, fcwb_ref, out_ref):
    B = out_ref.shape[0]
    nb = out_ref.shape[1]
    s_t = g_ref[0] + g_ref[1]
    ones_row = jnp.ones((1, nb), F32)
    dn_c = (((1,), (0,)), ((), ()))
    wgc = lax.dot_general(wih_ref[...], g2_ref[...], dn_c,
                          preferred_element_type=F32) + bihz_ref[...]
    w_hh = whh_ref[...]
    gh_bias = lax.dot_general(bhh_ref[...], ones_row, dn_c,
                              preferred_element_type=F32)
    hs = []
    for b in range(B):
        h = jnp.zeros((HT, nb), F32)
        for t in range(T):
            st = s_t[b * T + t:b * T + t + 1, :]
            st1 = jnp.concatenate([st, ones_row], axis=0)
            gi = lax.dot_general(wgc, st1, dn_c,
                                 preferred_element_type=F32)
            gh = lax.dot_general(w_hh, h, dn_c,
                                 preferred_element_type=F32) + gh_bias
            r = jax.nn.sigmoid(gi[:HT] + gh[:HT])
            z = jax.nn.sigmoid(gi[HT:2 * HT] + gh[HT:2 * HT])
            n = jnp.tanh(gi[2 * HT:] + r * gh[2 * HT:])
            h = (1.0 - z) * n + z * h
        hs.append(h)
        hs.append(ones_row)
    hcat = jnp.concatenate(hs, axis=0)
    pred = lax.dot_general(fcwb_ref[...], hcat, dn_c,
                           preferred_element_type=F32)
    out_ref[...] = pred


def kernel(x, edge_index, gcn_W, gcn_b, W_ih, W_hh, b_ih, b_hh, fc_W, fc_b):
    B, T, N = x.shape
    E = edge_index.shape[1]
    C = B * T
    HT = W_hh.shape[1]
    assert C == _LANES

    cht_a = -(-E // (_NS * _CHUNK))
    cht_a = 8 * (-(-cht_a // 8))
    tch = cht_a * _NS
    cht_c = tch // _NW
    epad = tch * _CHUNK
    npad = _NS * 128 * (-(-(N + 1) // (_NS * 128)))
    rows_t = npad // _NS

    src = edge_index[0]
    dst = edge_index[1]
    src_p = jnp.concatenate([src, jnp.zeros((epad - E,), I32)])
    dst_p = jnp.concatenate([dst, jnp.full((epad - E,), N, I32)])
    dst_r = dst_p.reshape(_NS, cht_a, _CHUNK)
    src_r = src_p.reshape(_NS, _NC, cht_c, _CHUNK).transpose(1, 0, 2, 3)

    x2 = jnp.pad(x.reshape(C, N), ((0, 0), (0, npad - N)))

    g, _unused_xp = _make_sc_kernel(npad, cht_a, cht_c, rows_t)(
        dst_r, src_r, x2)

    nb = npad // 8
    grid = npad // nb
    full = lambda shp: pl.BlockSpec(shp, lambda i: tuple(0 for _ in shp))
    g2 = jnp.stack([gcn_W[0], gcn_b], axis=1)
    bihz = jnp.pad(b_ih[:, None], ((0, 0), (1, 0)))
    fcrow = jnp.concatenate([fc_W[0], fc_b])
    fcwb = jnp.kron(jnp.eye(B, dtype=F32), fcrow[None, :])
    preds_pad = pl.pallas_call(
        functools.partial(_gru_body, T, HT),
        grid=(grid,),
        in_specs=[
            pl.BlockSpec((_NC, C, nb), lambda i: (0, 0, i)),
            full(g2.shape),
            full(W_ih.shape),
            full(W_hh.shape),
            full(bihz.shape),
            full((b_hh.shape[0], 1)),
            full(fcwb.shape),
        ],
        out_specs=pl.BlockSpec((B, nb), lambda i: (0, i)),
        out_shape=jax.ShapeDtypeStruct((B, npad), F32),
    )(g, g2, W_ih, W_hh, bihz, b_hh[:, None], fcwb)

    return preds_pad[:, :N]


# speedup vs baseline: 1.2548x; 1.2548x over previous
import functools

import jax
import jax.numpy as jnp
from jax import lax
from jax.experimental import pallas as pl
from jax.experimental.pallas import tpu as pltpu
from jax.experimental.pallas import tpu_sc as plsc

F32 = jnp.float32
I32 = jnp.int32

_NC = 2
_NS = 16
_NW = _NC * _NS
_LANES = 16
_CHUNK = 128


def _fast_rsqrt(d):
    i = plsc.bitcast(d, I32)
    i = jnp.int32(0x5F3759DF) - (i >> 1)
    y = plsc.bitcast(i, F32)
    for _ in range(3):
        y = y * (1.5 - 0.5 * d * y * y)
    return y


def _make_sc_kernel(npad, cht_a, cht_c, rows_t):
    mesh = plsc.VectorSubcoreMesh(core_axis_name="c", subcore_axis_name="s")
    lane_iota = lambda: lax.iota(I32, _LANES)

    @functools.partial(
        pl.kernel,
        out_type=jax.ShapeDtypeStruct((_NC, _LANES, npad), F32),
        mesh=mesh,
        compiler_params=pltpu.CompilerParams(use_tc_tiling_on_sc=False,
                                             needs_layout_passes=False),
        scratch_types=[
            pltpu.VMEM((cht_a, _CHUNK), I32),
            pltpu.VMEM((cht_c, _CHUNK), I32),
            pltpu.VMEM((_CHUNK,), F32),
            pltpu.VMEM((_LANES,), F32),
            pltpu.VMEM((8, _LANES), F32),
            pltpu.VMEM((_LANES, rows_t), F32),
            pltpu.VMEM((rows_t,), F32),
            pltpu.VMEM((rows_t,), F32),
            pltpu.VMEM((rows_t, _LANES), F32),
            pltpu.VMEM((_LANES, rows_t), F32),
            pltpu.VMEM((_CHUNK, _LANES), F32),
            pltpu.VMEM((_CHUNK, _LANES), F32),
            pltpu.VMEM((_CHUNK, _LANES), F32),
            pltpu.VMEM((_CHUNK, _LANES), F32),
            pltpu.VMEM_SHARED((npad,), F32),
            pltpu.VMEM_SHARED((npad, _LANES), F32),
            pltpu.VMEM_SHARED((npad, _LANES), F32),
            pltpu.SemaphoreType.DMA,
            pltpu.SemaphoreType.DMA,
            pltpu.SemaphoreType.DMA,
            pltpu.SemaphoreType.DMA,
            pltpu.SemaphoreType.DMA,
            pltpu.SemaphoreType.DMA,
            pltpu.SemaphoreType.DMA,
            pltpu.SemaphoreType.DMA,
            pltpu.SemaphoreType.DMA,
        ],
    )
    def sc_kernel(dst_h, src_h, x_h, g_h,
                  dst_v, src_v, ones_v, z1, z8, x_v, hist_v, dinv_v, xp_v,
                  gt_v, rb0, rb1, rb2, rb3, hist_sh, xp_sh, g_sh,
                  sem_a, g0, g1, g2, g3, s0, s1, s2, s3):
        c = lax.axis_index("c")
        s = lax.axis_index("s")
        for i in range(_CHUNK // _LANES):
            ones_v[pl.ds(i * _LANES, _LANES)] = jnp.ones((_LANES,), F32)
        z1[pl.ds(0, _LANES)] = jnp.zeros((_LANES,), F32)
        for i in range(8):
            z8[i, :] = jnp.zeros((_LANES,), F32)

        def zero_hist(j, _):
            pltpu.sync_copy(
                z1, hist_sh.at[pl.ds(s * rows_t + j * _LANES, _LANES)])
            return 0

        lax.fori_loop(0, rows_t // _LANES, zero_hist, 0)
        plsc.subcore_barrier()

        scope_a = jax.named_scope("phaseA")
        scope_a.__enter__()
        pltpu.sync_copy(dst_h.at[s], dst_v)
        n_grp = -(-cht_a // 8)

        def hist_fire(j, _):
            pltpu.async_copy(ones_v, hist_sh.at[dst_v.at[j]], sem_a,
                             add=True)
            return 0

        lax.fori_loop(0, cht_a, hist_fire, 0)

        def hist_drain(j, _):
            pltpu.make_async_copy(
                ones_v, hist_sh.at[dst_v.at[0]], sem_a).wait()
            return 0

        lax.fori_loop(0, cht_a, hist_drain, 0)
        plsc.subcore_barrier()

        scope_a.__exit__(None, None, None)
        scope_b = jax.named_scope("phaseB")
        scope_b.__enter__()
        pltpu.sync_copy(hist_sh.at[pl.ds(s * rows_t, rows_t)], hist_v)
        pltpu.sync_copy(x_h.at[:, pl.ds(s * rows_t, rows_t)], x_v)

        def brow(q, _):
            d = hist_v[pl.ds(q * _LANES, _LANES)] + 1.0
            dv = _fast_rsqrt(d)
            dinv_v[pl.ds(q * _LANES, _LANES)] = dv
            idx_row = q * _LANES + lane_iota()
            for ch in range(_LANES):
                idx_col = jnp.full((_LANES,), ch, I32)
                xcol = x_v[ch, pl.ds(q * _LANES, _LANES)]
                plsc.store_scatter(xp_v, [idx_row, idx_col], xcol * dv)
            return 0

        lax.fori_loop(0, rows_t // _LANES, brow, 0)
        pltpu.sync_copy(xp_v, xp_sh.at[pl.ds(s * rows_t, rows_t)])

        @pl.when(c == 0)
        def _():
            pltpu.sync_copy(xp_v, g_sh.at[pl.ds(s * rows_t, rows_t)])

        @pl.when(c == 1)
        def _():
            def zg(j, _):
                pltpu.sync_copy(z8, g_sh.at[pl.ds(s * rows_t + j * 8, 8)])
                return 0

            lax.fori_loop(0, rows_t // 8, zg, 0)

        plsc.subcore_barrier()

        scope_b.__exit__(None, None, None)
        scope_c = jax.named_scope("phaseC")
        scope_c.__enter__()
        pltpu.sync_copy(src_h.at[c, s], src_v)
        coff = c * cht_c
        bufs = (rb0, rb1, rb2, rb3)
        gsems = (g0, g1, g2, g3)
        ssems = (s0, s1, s2, s3)
        xp_t = xp_sh
        for k in range(4):
            pltpu.async_copy(xp_t.at[src_v.at[k]], bufs[k], gsems[k])

        def edge_grp(gi_, _):
            for k in range(4):
                j = gi_ * 4 + k
                pltpu.make_async_copy(
                    xp_t.at[src_v.at[j]], bufs[k], gsems[k]).wait()
                pltpu.async_copy(bufs[k], g_sh.at[dst_v.at[coff + j]],
                                 ssems[k], add=True)
            for k in range(4):
                jn = gi_ * 4 + k + 4

                @pl.when(jn < cht_c)
                def _():
                    pltpu.make_async_copy(
                        bufs[k], g_sh.at[dst_v.at[coff + jn - 4]],
                        ssems[k]).wait()
                    pltpu.async_copy(xp_t.at[src_v.at[jn]], bufs[k],
                                     gsems[k])

            return 0

        lax.fori_loop(0, cht_c // 4, edge_grp, 0)
        for k in range(4):
            pltpu.make_async_copy(
                bufs[k], g_sh.at[dst_v.at[coff + cht_c - 4 + k]],
                ssems[k]).wait()
        plsc.subcore_barrier()

        scope_c.__exit__(None, None, None)
        scope_d = jax.named_scope("phaseD")
        scope_d.__enter__()
        pltpu.sync_copy(g_sh.at[pl.ds(s * rows_t, rows_t)], xp_v)

        def drow(q, _):
            dvv = dinv_v[pl.ds(q * _LANES, _LANES)]
            idx_row = q * _LANES + lane_iota()
            for ch in range(_LANES):
                idx_col = jnp.full((_LANES,), ch, I32)
                col = plsc.load_gather(xp_v, [idx_row, idx_col]) * dvv
                gt_v[ch, pl.ds(q * _LANES, _LANES)] = col
            return 0

        lax.fori_loop(0, rows_t // _LANES, drow, 0)
        pltpu.sync_copy(gt_v, g_h.at[c, :, pl.ds(s * rows_t, rows_t)])
        scope_d.__exit__(None, None, None)

    return sc_kernel


def _gru_body(T, HT, g_ref, g2_ref, wih_ref, whh_ref, bihz_ref,
              bhh_ref, fcwb_ref, out_ref):
    B = out_ref.shape[0]
    nb = out_ref.shape[1]
    s_t = g_ref[0] + g_ref[1]
    ones_row = jnp.ones((1, nb), F32)
    dn_c = (((1,), (0,)), ((), ()))
    wgc = lax.dot_general(wih_ref[...], g2_ref[...], dn_c,
                          preferred_element_type=F32) + bihz_ref[...]
    w_hh = whh_ref[...]
    gh_bias = lax.dot_general(bhh_ref[...], ones_row, dn_c,
                              preferred_element_type=F32)
    hs = []
    for b in range(B):
        h = jnp.zeros((HT, nb), F32)
        for t in range(T):
            st = s_t[b * T + t:b * T + t + 1, :]
            st1 = jnp.concatenate([st, ones_row], axis=0)
            gi = lax.dot_general(wgc, st1, dn_c,
                                 preferred_element_type=F32)
            gh = lax.dot_general(w_hh, h, dn_c,
                                 preferred_element_type=F32) + gh_bias
            r = jax.nn.sigmoid(gi[:HT] + gh[:HT])
            z = jax.nn.sigmoid(gi[HT:2 * HT] + gh[HT:2 * HT])
            n = jnp.tanh(gi[2 * HT:] + r * gh[2 * HT:])
            h = (1.0 - z) * n + z * h
        hs.append(h)
        hs.append(ones_row)
    hcat = jnp.concatenate(hs, axis=0)
    pred = lax.dot_general(fcwb_ref[...], hcat, dn_c,
                           preferred_element_type=F32)
    out_ref[...] = pred


def kernel(x, edge_index, gcn_W, gcn_b, W_ih, W_hh, b_ih, b_hh, fc_W, fc_b):
    B, T, N = x.shape
    E = edge_index.shape[1]
    C = B * T
    HT = W_hh.shape[1]
    assert C == _LANES

    cht_a = -(-E // (_NS * _CHUNK))
    cht_a = 8 * (-(-cht_a // 8))
    tch = cht_a * _NS
    cht_c = tch // _NW
    epad = tch * _CHUNK
    npad = _NS * 128 * (-(-(N + 1) // (_NS * 128)))
    rows_t = npad // _NS

    src = edge_index[0]
    dst = edge_index[1]
    src_p = jnp.concatenate([src, jnp.zeros((epad - E,), I32)])
    dst_p = jnp.concatenate([dst, jnp.full((epad - E,), N, I32)])
    dst_r = dst_p.reshape(_NS, cht_a, _CHUNK)
    src_r = src_p.reshape(_NS, _NC, cht_c, _CHUNK).transpose(1, 0, 2, 3)

    x2 = jnp.pad(x.reshape(C, N), ((0, 0), (0, npad - N)))

    g = _make_sc_kernel(npad, cht_a, cht_c, rows_t)(dst_r, src_r, x2)

    nb = npad // 8
    grid = npad // nb
    full = lambda shp: pl.BlockSpec(shp, lambda i: tuple(0 for _ in shp))
    g2 = jnp.stack([gcn_W[0], gcn_b], axis=1)
    bihz = jnp.pad(b_ih[:, None], ((0, 0), (1, 0)))
    fcrow = jnp.concatenate([fc_W[0], fc_b])
    fcwb = jnp.kron(jnp.eye(B, dtype=F32), fcrow[None, :])
    preds_pad = pl.pallas_call(
        functools.partial(_gru_body, T, HT),
        grid=(grid,),
        in_specs=[
            pl.BlockSpec((_NC, C, nb), lambda i: (0, 0, i)),
            full(g2.shape),
            full(W_ih.shape),
            full(W_hh.shape),
            full(bihz.shape),
            full((b_hh.shape[0], 1)),
            full(fcwb.shape),
        ],
        out_specs=pl.BlockSpec((B, nb), lambda i: (0, i)),
        out_shape=jax.ShapeDtypeStruct((B, npad), F32),
    )(g, g2, W_ih, W_hh, bihz, b_hh[:, None], fcwb)

    return preds_pad[:, :N]


# speedup vs baseline: 1.2964x; 1.0331x over previous
import functools

import jax
import jax.numpy as jnp
from jax import lax
from jax.experimental import pallas as pl
from jax.experimental.pallas import tpu as pltpu
from jax.experimental.pallas import tpu_sc as plsc

F32 = jnp.float32
I32 = jnp.int32

_NC = 2
_NS = 16
_NW = _NC * _NS
_LANES = 16
_CHUNK = 128


def _fast_rsqrt(d):
    i = plsc.bitcast(d, I32)
    i = jnp.int32(0x5F3759DF) - (i >> 1)
    y = plsc.bitcast(i, F32)
    for _ in range(3):
        y = y * (1.5 - 0.5 * d * y * y)
    return y


def _make_sc_kernel(npad, cht_a, cht_c, rows_t):
    mesh = plsc.VectorSubcoreMesh(core_axis_name="c", subcore_axis_name="s")
    lane_iota = lambda: lax.iota(I32, _LANES)

    @functools.partial(
        pl.kernel,
        out_type=jax.ShapeDtypeStruct((_NC, _LANES, npad), F32),
        mesh=mesh,
        compiler_params=pltpu.CompilerParams(use_tc_tiling_on_sc=False,
                                             needs_layout_passes=False),
        scratch_types=[
            pltpu.VMEM((cht_a, _CHUNK), I32),
            pltpu.VMEM((cht_c, _CHUNK), I32),
            pltpu.VMEM((_CHUNK,), F32),
            pltpu.VMEM((_LANES,), F32),
            pltpu.VMEM((8, _LANES), F32),
            pltpu.VMEM((_LANES, rows_t), F32),
            pltpu.VMEM((rows_t,), F32),
            pltpu.VMEM((rows_t,), F32),
            pltpu.VMEM((rows_t, _LANES), F32),
            pltpu.VMEM((_LANES, rows_t), F32),
            pltpu.VMEM((_CHUNK, _LANES), F32),
            pltpu.VMEM((_CHUNK, _LANES), F32),
            pltpu.VMEM((_CHUNK, _LANES), F32),
            pltpu.VMEM((_CHUNK, _LANES), F32),
            pltpu.VMEM_SHARED((npad,), F32),
            pltpu.VMEM_SHARED((npad, _LANES), F32),
            pltpu.VMEM_SHARED((npad, _LANES), F32),
            pltpu.SemaphoreType.DMA,
            pltpu.SemaphoreType.DMA,
            pltpu.SemaphoreType.DMA,
            pltpu.SemaphoreType.DMA,
            pltpu.SemaphoreType.DMA,
            pltpu.SemaphoreType.DMA,
            pltpu.SemaphoreType.DMA,
            pltpu.SemaphoreType.DMA,
            pltpu.SemaphoreType.DMA,
        ],
    )
    def sc_kernel(dst_h, src_h, x_h, g_h,
                  dst_v, src_v, ones_v, z1, z8, x_v, hist_v, dinv_v, xp_v,
                  gt_v, rb0, rb1, rb2, rb3, hist_sh, xp_sh, g_sh,
                  sem_a, g0, g1, g2, g3, s0, s1, s2, s3):
        c = lax.axis_index("c")
        s = lax.axis_index("s")
        pltpu.async_copy(dst_h.at[s], dst_v, g0)
        pltpu.async_copy(x_h.at[:, pl.ds(s * rows_t, rows_t)], x_v, g1)
        pltpu.async_copy(src_h.at[s * _NC + c], src_v, g2)
        for i in range(_CHUNK // _LANES):
            ones_v[pl.ds(i * _LANES, _LANES)] = jnp.ones((_LANES,), F32)
        z1[pl.ds(0, _LANES)] = jnp.zeros((_LANES,), F32)
        for i in range(8):
            z8[i, :] = jnp.zeros((_LANES,), F32)

        def zero_hist(j, _):
            pltpu.sync_copy(
                z1, hist_sh.at[pl.ds(s * rows_t + j * _LANES, _LANES)])
            return 0

        lax.fori_loop(0, rows_t // _LANES, zero_hist, 0)
        plsc.subcore_barrier()

        scope_a = jax.named_scope("phaseA")
        scope_a.__enter__()
        pltpu.make_async_copy(dst_h.at[s], dst_v, g0).wait()

        def hist_fire(j, _):
            pltpu.async_copy(ones_v, hist_sh.at[dst_v.at[j]], sem_a,
                             add=True)
            return 0

        lax.fori_loop(0, cht_a, hist_fire, 0)

        def hist_drain(j, _):
            pltpu.make_async_copy(
                ones_v, hist_sh.at[dst_v.at[0]], sem_a).wait()
            return 0

        lax.fori_loop(0, cht_a, hist_drain, 0)
        plsc.subcore_barrier()

        scope_a.__exit__(None, None, None)
        scope_b = jax.named_scope("phaseB")
        scope_b.__enter__()
        pltpu.sync_copy(hist_sh.at[pl.ds(s * rows_t, rows_t)], hist_v)
        pltpu.make_async_copy(
            x_h.at[:, pl.ds(s * rows_t, rows_t)], x_v, g1).wait()

        def brow(q, _):
            d = hist_v[pl.ds(q * _LANES, _LANES)] + 1.0
            dv = _fast_rsqrt(d)
            dinv_v[pl.ds(q * _LANES, _LANES)] = dv
            idx_row = q * _LANES + lane_iota()
            for ch in range(_LANES):
                idx_col = jnp.full((_LANES,), ch, I32)
                xcol = x_v[ch, pl.ds(q * _LANES, _LANES)]
                plsc.store_scatter(xp_v, [idx_row, idx_col], xcol * dv)
            return 0

        lax.fori_loop(0, rows_t // _LANES, brow, 0)
        pltpu.sync_copy(xp_v, xp_sh.at[pl.ds(s * rows_t, rows_t)])

        @pl.when(c == 0)
        def _():
            pltpu.sync_copy(xp_v, g_sh.at[pl.ds(s * rows_t, rows_t)])

        @pl.when(c == 1)
        def _():
            def zg(j, _):
                pltpu.sync_copy(z8, g_sh.at[pl.ds(s * rows_t + j * 8, 8)])
                return 0

            lax.fori_loop(0, rows_t // 8, zg, 0)

        plsc.subcore_barrier()

        scope_b.__exit__(None, None, None)
        scope_c = jax.named_scope("phaseC")
        scope_c.__enter__()
        pltpu.make_async_copy(src_h.at[s * _NC + c], src_v, g2).wait()
        coff = c * cht_c
        bufs = (rb0, rb1, rb2, rb3)
        gsems = (g0, g1, g2, g3)
        ssems = (s0, s1, s2, s3)
        xp_t = xp_sh
        for k in range(4):
            pltpu.async_copy(xp_t.at[src_v.at[k]], bufs[k], gsems[k])

        def edge_grp(gi_, _):
            for k in range(4):
                j = gi_ * 4 + k
                pltpu.make_async_copy(
                    xp_t.at[src_v.at[j]], bufs[k], gsems[k]).wait()
                pltpu.async_copy(bufs[k], g_sh.at[dst_v.at[coff + j]],
                                 ssems[k], add=True)
            for k in range(4):
                jn = gi_ * 4 + k + 4

                @pl.when(jn < cht_c)
                def _():
                    pltpu.make_async_copy(
                        bufs[k], g_sh.at[dst_v.at[coff + jn - 4]],
                        ssems[k]).wait()
                    pltpu.async_copy(xp_t.at[src_v.at[jn]], bufs[k],
                                     gsems[k])

            return 0

        lax.fori_loop(0, cht_c // 4, edge_grp, 0)
        for k in range(4):
            pltpu.make_async_copy(
                bufs[k], g_sh.at[dst_v.at[coff + cht_c - 4 + k]],
                ssems[k]).wait()
        plsc.subcore_barrier()

        scope_c.__exit__(None, None, None)
        scope_d = jax.named_scope("phaseD")
        scope_d.__enter__()
        pltpu.sync_copy(g_sh.at[pl.ds(s * rows_t, rows_t)], xp_v)

        def drow(q, _):
            dvv = dinv_v[pl.ds(q * _LANES, _LANES)]
            idx_row = q * _LANES + lane_iota()
            for ch in range(_LANES):
                idx_col = jnp.full((_LANES,), ch, I32)
                col = plsc.load_gather(xp_v, [idx_row, idx_col]) * dvv
                gt_v[ch, pl.ds(q * _LANES, _LANES)] = col
            return 0

        lax.fori_loop(0, rows_t // _LANES, drow, 0)
        pltpu.sync_copy(gt_v, g_h.at[c, :, pl.ds(s * rows_t, rows_t)])
        scope_d.__exit__(None, None, None)

    return sc_kernel


def _gru_body(T, HT, g_ref, g2_ref, wih_ref, whh_ref, bihz_ref,
              bhh_ref, fcwb_ref, out_ref):
    B = out_ref.shape[0]
    nb = out_ref.shape[1]
    s_t = g_ref[0] + g_ref[1]
    ones_row = jnp.ones((1, nb), F32)
    dn_c = (((1,), (0,)), ((), ()))
    wgc = lax.dot_general(wih_ref[...], g2_ref[...], dn_c,
                          preferred_element_type=F32) + bihz_ref[...]
    w_hh = whh_ref[...]
    gh_bias = lax.dot_general(bhh_ref[...], ones_row, dn_c,
                              preferred_element_type=F32)
    hs = []
    for b in range(B):
        h = jnp.zeros((HT, nb), F32)
        for t in range(T):
            st = s_t[b * T + t:b * T + t + 1, :]
            st1 = jnp.concatenate([st, ones_row], axis=0)
            gi = lax.dot_general(wgc, st1, dn_c,
                                 preferred_element_type=F32)
            gh = lax.dot_general(w_hh, h, dn_c,
                                 preferred_element_type=F32) + gh_bias
            r = jax.nn.sigmoid(gi[:HT] + gh[:HT])
            z = jax.nn.sigmoid(gi[HT:2 * HT] + gh[HT:2 * HT])
            n = jnp.tanh(gi[2 * HT:] + r * gh[2 * HT:])
            h = (1.0 - z) * n + z * h
        hs.append(h)
        hs.append(ones_row)
    hcat = jnp.concatenate(hs, axis=0)
    pred = lax.dot_general(fcwb_ref[...], hcat, dn_c,
                           preferred_element_type=F32)
    out_ref[...] = pred


def kernel(x, edge_index, gcn_W, gcn_b, W_ih, W_hh, b_ih, b_hh, fc_W, fc_b):
    B, T, N = x.shape
    E = edge_index.shape[1]
    C = B * T
    HT = W_hh.shape[1]
    assert C == _LANES

    cht_a = -(-E // (_NS * _CHUNK))
    cht_a = 8 * (-(-cht_a // 8))
    tch = cht_a * _NS
    cht_c = tch // _NW
    epad = tch * _CHUNK
    npad = _NS * 128 * (-(-(N + 1) // (_NS * 128)))
    rows_t = npad // _NS

    src = edge_index[0]
    dst = edge_index[1]
    src_p = jnp.concatenate([src, jnp.zeros((epad - E,), I32)])
    dst_p = jnp.concatenate([dst, jnp.full((epad - E,), N, I32)])
    dst_r = dst_p.reshape(_NS, cht_a, _CHUNK)
    src_r = src_p.reshape(_NW, cht_c, _CHUNK)

    x2 = jnp.pad(x.reshape(C, N), ((0, 0), (0, npad - N)))

    g = _make_sc_kernel(npad, cht_a, cht_c, rows_t)(dst_r, src_r, x2)

    nb = npad // 8
    grid = npad // nb
    full = lambda shp: pl.BlockSpec(shp, lambda i: tuple(0 for _ in shp))
    g2 = jnp.stack([gcn_W[0], gcn_b], axis=1)
    bihz = jnp.pad(b_ih[:, None], ((0, 0), (1, 0)))
    fcrow = jnp.concatenate([fc_W[0], fc_b])
    fcwb = jnp.kron(jnp.eye(B, dtype=F32), fcrow[None, :])
    preds_pad = pl.pallas_call(
        functools.partial(_gru_body, T, HT),
        grid=(grid,),
        in_specs=[
            pl.BlockSpec((_NC, C, nb), lambda i: (0, 0, i)),
            full(g2.shape),
            full(W_ih.shape),
            full(W_hh.shape),
            full(bihz.shape),
            full((b_hh.shape[0], 1)),
            full(fcwb.shape),
        ],
        out_specs=pl.BlockSpec((B, nb), lambda i: (0, i)),
        out_shape=jax.ShapeDtypeStruct((B, npad), F32),
    )(g, g2, W_ih, W_hh, bihz, b_hh[:, None], fcwb)

    return preds_pad[:, :N]
